# baseline (speedup 1.0000x reference)
import jax
import jax.numpy as jnp
from jax import lax
from jax.experimental import pallas as pl
from jax.experimental.pallas import tpu as pltpu
from jax.experimental.pallas import tpu_sc as plsc

_T = 256
_TG = 512
_NW = 32


def _gate_body(x_ref, wg_ref, bg_ref, tri_ref, s_ref, eid_ref, grank_ref,
               ccnt_ref, acc_ref):
    pid = pl.program_id(0)

    @pl.when(pid == 0)
    def _():
        acc_ref[...] = jnp.zeros_like(acc_ref)

    s = jnp.dot(x_ref[...], wg_ref[...],
                preferred_element_type=jnp.float32) + bg_ref[...]
    s_ref[...] = s
    lane = lax.broadcasted_iota(jnp.int32, (_TG, 128), 1)
    sm = jnp.where(lane < 8, s, -jnp.inf)
    mx = jnp.max(sm, axis=1, keepdims=True)
    eid = jnp.min(jnp.where(sm == mx, lane, 128), axis=1).astype(jnp.int32)
    oh = (lane == eid[:, None]).astype(jnp.bfloat16)
    rkm = jnp.dot(tri_ref[...], oh, preferred_element_type=jnp.float32)
    ohf = oh.astype(jnp.float32)
    rank = jnp.sum(rkm * ohf, axis=1)
    base = jnp.sum(acc_ref[0:1, :].astype(jnp.float32) * ohf, axis=1)
    grank_ref[...] = (rank + base).astype(jnp.int32).reshape(1, 1, _TG)
    eid_ref[...] = eid.reshape(1, 1, _TG)
    blk2 = jnp.sum(ohf, axis=0, keepdims=True).astype(jnp.int32)
    newacc = acc_ref[0:1, :] + blk2
    acc_ref[0:1, :] = newacc
    ccnt_ref[...] = newacc.reshape(1, 1, 128)


def _gate(xf, Wg, bg):
    N, D = xf.shape
    E = Wg.shape[1]
    NB = N // _TG
    wg_pad = jnp.zeros((D, 128), Wg.dtype).at[:, :E].set(Wg)
    bg_pad = jnp.zeros((1, 128), jnp.float32).at[0, :E].set(bg)
    r = lax.broadcasted_iota(jnp.int32, (_TG, _TG), 0)
    c = lax.broadcasted_iota(jnp.int32, (_TG, _TG), 1)
    tri = (r > c).astype(jnp.bfloat16)
    scores, eid3, grank3, ccnt3 = pl.pallas_call(
        _gate_body,
        grid=(NB,),
        in_specs=[
            pl.BlockSpec((_TG, D), lambda i: (i, 0)),
            pl.BlockSpec((D, 128), lambda i: (0, 0)),
            pl.BlockSpec((1, 128), lambda i: (0, 0)),
            pl.BlockSpec((_TG, _TG), lambda i: (0, 0)),
        ],
        out_specs=[
            pl.BlockSpec((_TG, 128), lambda i: (i, 0)),
            pl.BlockSpec((1, 1, _TG), lambda i: (i, 0, 0)),
            pl.BlockSpec((1, 1, _TG), lambda i: (i, 0, 0)),
            pl.BlockSpec((1, 1, 128), lambda i: (i, 0, 0)),
        ],
        out_shape=[
            jax.ShapeDtypeStruct((N, 128), jnp.float32),
            jax.ShapeDtypeStruct((NB, 1, _TG), jnp.int32),
            jax.ShapeDtypeStruct((NB, 1, _TG), jnp.int32),
            jax.ShapeDtypeStruct((NB, 1, 128), jnp.int32),
        ],
        scratch_shapes=[pltpu.VMEM((8, 128), jnp.int32)],
    )(xf, wg_pad, bg_pad, tri)
    return (scores, eid3.reshape(N), grank3.reshape(N),
            ccnt3[-1, 0, :E])


def _ffn_body(pref_ref, xs_ref, w1_ref, b1_ref, w2_ref, b2_ref, y_ref):
    @pl.when(pref_ref[1, pl.program_id(0)] == 1)
    def _():
        xv = xs_ref[...].astype(jnp.bfloat16)
        h = jnp.dot(xv, w1_ref[0].astype(jnp.bfloat16),
                    preferred_element_type=jnp.float32)
        h = jnp.maximum(h + b1_ref[0, 0], 0.0)
        y = jnp.dot(h.astype(jnp.bfloat16), w2_ref[0].astype(jnp.bfloat16),
                    preferred_element_type=jnp.float32)
        y_ref[...] = y + b2_ref[0, 0]


def _ffn(xs, pref, W1, b1, W2, b2):
    NPAD, D = xs.shape
    E, _, H = W1.shape
    NT = NPAD // _T
    grid_spec = pltpu.PrefetchScalarGridSpec(
        num_scalar_prefetch=1,
        grid=(NT,),
        in_specs=[
            pl.BlockSpec((_T, D), lambda i, p: (i, 0)),
            pl.BlockSpec((1, D, H), lambda i, p: (p[0, i], 0, 0)),
            pl.BlockSpec((1, 1, H), lambda i, p: (p[0, i], 0, 0)),
            pl.BlockSpec((1, H, D), lambda i, p: (p[0, i], 0, 0)),
            pl.BlockSpec((1, 1, D), lambda i, p: (p[0, i], 0, 0)),
        ],
        out_specs=pl.BlockSpec((_T, D), lambda i, p: (i, 0)),
    )
    return pl.pallas_call(
        _ffn_body,
        grid_spec=grid_spec,
        out_shape=jax.ShapeDtypeStruct((NPAD, D), jnp.float32),
    )(pref, xs, W1, b1.reshape(E, 1, H), W2, b2.reshape(E, 1, D))


def _sc_mesh():
    return plsc.VectorSubcoreMesh(core_axis_name="c", subcore_axis_name="s")


def _sc_gather(table, idx):
    V, D = table.shape
    B = idx.shape[0]
    bpw = B // _NW
    nch = 4
    ch = bpw // nch

    def body(tab_hbm, idx_hbm, out_hbm, idx_v, b0, b1, gsem, osem):
        wid = lax.axis_index("s") * 2 + lax.axis_index("c")
        base = wid * bpw
        pltpu.sync_copy(idx_hbm.at[pl.ds(base, bpw)], idx_v)
        bufs = (b0, b1)
        g = [None] * nch
        o = [None] * nch
        g[0] = pltpu.async_copy(tab_hbm.at[idx_v.at[pl.ds(0, ch)]], bufs[0],
                                gsem.at[0])
        for c in range(nch):
            b = c % 2
            g[c].wait()
            if c >= 1:
                o[c - 1].wait()
            o[c] = pltpu.async_copy(bufs[b], out_hbm.at[pl.ds(base + c * ch, ch)],
                                    osem.at[b])
            if c + 1 < nch:
                g[c + 1] = pltpu.async_copy(
                    tab_hbm.at[idx_v.at[pl.ds((c + 1) * ch, ch)]],
                    bufs[1 - b], gsem.at[1 - b])
        o[nch - 1].wait()

    return pl.kernel(
        body,
        out_type=jax.ShapeDtypeStruct((B, D), table.dtype),
        mesh=_sc_mesh(),
        scratch_types=[
            pltpu.VMEM((bpw,), jnp.int32),
            pltpu.VMEM((ch, D), table.dtype),
            pltpu.VMEM((ch, D), table.dtype),
            pltpu.SemaphoreType.DMA((2,)),
            pltpu.SemaphoreType.DMA((2,)),
        ],
    )(table, idx)


def _sc_scatter(rows, idx, V):
    B, D = rows.shape
    bpw = B // _NW
    nch = 4
    ch = bpw // nch
    idx2 = idx.reshape(_NW * nch, ch)

    def body(rows_hbm, idx_hbm, out_hbm, idx_v, b0, b1, rsem, wsem):
        wid = lax.axis_index("s") * 2 + lax.axis_index("c")
        base = wid * bpw
        pltpu.sync_copy(idx_hbm.at[pl.ds(wid * nch, nch)], idx_v)
        bufs = (b0, b1)
        r = [None] * nch
        w = [None] * nch
        r[0] = pltpu.async_copy(rows_hbm.at[pl.ds(base, ch)], bufs[0],
                                rsem.at[0])
        for c in range(nch):
            b = c % 2
            r[c].wait()
            if c >= 1:
                w[c - 1].wait()
            w[c] = pltpu.async_copy(bufs[b], out_hbm.at[idx_v.at[c]],
                                    wsem.at[b])
            if c + 1 < nch:
                r[c + 1] = pltpu.async_copy(
                    rows_hbm.at[pl.ds(base + (c + 1) * ch, ch)],
                    bufs[1 - b], rsem.at[1 - b])
        w[nch - 1].wait()

    return pl.kernel(
        body,
        out_type=jax.ShapeDtypeStruct((V, D), rows.dtype),
        mesh=_sc_mesh(),
        scratch_types=[
            pltpu.VMEM((nch, ch), jnp.int32),
            pltpu.VMEM((ch, D), rows.dtype),
            pltpu.VMEM((ch, D), rows.dtype),
            pltpu.SemaphoreType.DMA((2,)),
            pltpu.SemaphoreType.DMA((2,)),
        ],
    )(rows, idx2)


def kernel(x, Wg, bg, W1, b1, W2, b2):
    B, S, D = x.shape
    E = Wg.shape[1]
    N = B * S
    NT = N // _T + E
    NPAD = NT * _T

    xf = x.reshape(N, D)
    scores, eid, grank, counts = _gate(xf, Wg, bg)

    tiles_per_e = (counts + _T - 1) // _T
    tile_bound = jnp.cumsum(tiles_per_e).astype(jnp.int32)
    pad_off = jnp.concatenate([jnp.zeros((1,), jnp.int32),
                               tile_bound[:-1]]) * _T
    pos = pad_off[eid] + grank
    t_idx = jnp.arange(NT, dtype=jnp.int32)
    tile_eid = jnp.sum((t_idx[:, None] >= tile_bound[None, :]).astype(
        jnp.int32), axis=1)
    valid = (tile_eid < E).astype(jnp.int32)
    pref = jnp.stack([jnp.minimum(tile_eid, E - 1), valid])

    xs = _sc_scatter(xf, pos, NPAD)
    ys = _ffn(xs, pref, W1, b1, W2, b2)
    out = _sc_gather(ys, pos)
    return (out.reshape(B, S, D),
            scores[:, :E].reshape(B, S, E))


# speedup vs baseline: 1.0435x; 1.0435x over previous
import jax
import jax.numpy as jnp
from jax import lax
from jax.experimental import pallas as pl
from jax.experimental.pallas import tpu as pltpu
from jax.experimental.pallas import tpu_sc as plsc

_T = 256
_TG = 512
_NW = 32


def _gate_body(x_ref, wg_ref, bg_ref, s_ref):
    s_ref[...] = jnp.dot(x_ref[...], wg_ref[...],
                         preferred_element_type=jnp.float32) + bg_ref[...]


def _gate(xf, Wg, bg):
    N, D = xf.shape
    E = Wg.shape[1]
    NB = N // _TG
    wg_pad = jnp.zeros((D, 128), Wg.dtype).at[:, :E].set(Wg)
    bg_pad = jnp.zeros((1, 128), jnp.float32).at[0, :E].set(bg)
    return pl.pallas_call(
        _gate_body,
        grid=(NB,),
        in_specs=[
            pl.BlockSpec((_TG, D), lambda i: (i, 0)),
            pl.BlockSpec((D, 128), lambda i: (0, 0)),
            pl.BlockSpec((1, 128), lambda i: (0, 0)),
        ],
        out_specs=pl.BlockSpec((_TG, 128), lambda i: (i, 0)),
        out_shape=jax.ShapeDtypeStruct((N, 128), jnp.float32),
    )(xf, wg_pad, bg_pad)


def _ffn_body(pref_ref, xs_ref, w1_ref, b1_ref, w2_ref, b2_ref, y_ref):
    @pl.when(pref_ref[1, pl.program_id(0)] == 1)
    def _():
        xv = xs_ref[...].astype(jnp.bfloat16)
        h = jnp.dot(xv, w1_ref[0].astype(jnp.bfloat16),
                    preferred_element_type=jnp.float32)
        h = jnp.maximum(h + b1_ref[0, 0], 0.0)
        y = jnp.dot(h.astype(jnp.bfloat16), w2_ref[0].astype(jnp.bfloat16),
                    preferred_element_type=jnp.float32)
        y_ref[...] = y + b2_ref[0, 0]


def _ffn(xs, pref, W1, b1, W2, b2):
    NPAD, D = xs.shape
    E, _, H = W1.shape
    NT = NPAD // _T
    grid_spec = pltpu.PrefetchScalarGridSpec(
        num_scalar_prefetch=1,
        grid=(NT,),
        in_specs=[
            pl.BlockSpec((_T, D), lambda i, p: (i, 0)),
            pl.BlockSpec((1, D, H), lambda i, p: (p[0, i], 0, 0)),
            pl.BlockSpec((1, 1, H), lambda i, p: (p[0, i], 0, 0)),
            pl.BlockSpec((1, H, D), lambda i, p: (p[0, i], 0, 0)),
            pl.BlockSpec((1, 1, D), lambda i, p: (p[0, i], 0, 0)),
        ],
        out_specs=pl.BlockSpec((_T, D), lambda i, p: (i, 0)),
    )
    return pl.pallas_call(
        _ffn_body,
        grid_spec=grid_spec,
        out_shape=jax.ShapeDtypeStruct((NPAD, D), jnp.float32),
    )(pref, xs, W1, b1.reshape(E, 1, H), W2, b2.reshape(E, 1, D))


def _sc_mesh():
    return plsc.VectorSubcoreMesh(core_axis_name="c", subcore_axis_name="s")


def _sc_gather(table, idx):
    V, D = table.shape
    B = idx.shape[0]
    bpw = B // _NW
    nch = 4
    ch = bpw // nch

    def body(tab_hbm, idx_hbm, out_hbm, idx_v, b0, b1, gsem, osem):
        wid = lax.axis_index("s") * 2 + lax.axis_index("c")
        base = wid * bpw
        pltpu.sync_copy(idx_hbm.at[pl.ds(base, bpw)], idx_v)
        bufs = (b0, b1)
        g = [None] * nch
        o = [None] * nch
        g[0] = pltpu.async_copy(tab_hbm.at[idx_v.at[pl.ds(0, ch)]], bufs[0],
                                gsem.at[0])
        for c in range(nch):
            b = c % 2
            g[c].wait()
            if c >= 1:
                o[c - 1].wait()
            o[c] = pltpu.async_copy(bufs[b], out_hbm.at[pl.ds(base + c * ch, ch)],
                                    osem.at[b])
            if c + 1 < nch:
                g[c + 1] = pltpu.async_copy(
                    tab_hbm.at[idx_v.at[pl.ds((c + 1) * ch, ch)]],
                    bufs[1 - b], gsem.at[1 - b])
        o[nch - 1].wait()

    return pl.kernel(
        body,
        out_type=jax.ShapeDtypeStruct((B, D), table.dtype),
        mesh=_sc_mesh(),
        scratch_types=[
            pltpu.VMEM((bpw,), jnp.int32),
            pltpu.VMEM((ch, D), table.dtype),
            pltpu.VMEM((ch, D), table.dtype),
            pltpu.SemaphoreType.DMA((2,)),
            pltpu.SemaphoreType.DMA((2,)),
        ],
    )(table, idx)


def _sc_scatter(rows, idx, V):
    B, D = rows.shape
    bpw = B // _NW
    nch = 4
    ch = bpw // nch
    idx2 = idx.reshape(_NW * nch, ch)

    def body(rows_hbm, idx_hbm, out_hbm, idx_v, b0, b1, rsem, wsem):
        wid = lax.axis_index("s") * 2 + lax.axis_index("c")
        base = wid * bpw
        pltpu.sync_copy(idx_hbm.at[pl.ds(wid * nch, nch)], idx_v)
        bufs = (b0, b1)
        r = [None] * nch
        w = [None] * nch
        r[0] = pltpu.async_copy(rows_hbm.at[pl.ds(base, ch)], bufs[0],
                                rsem.at[0])
        for c in range(nch):
            b = c % 2
            r[c].wait()
            if c >= 1:
                w[c - 1].wait()
            w[c] = pltpu.async_copy(bufs[b], out_hbm.at[idx_v.at[c]],
                                    wsem.at[b])
            if c + 1 < nch:
                r[c + 1] = pltpu.async_copy(
                    rows_hbm.at[pl.ds(base + (c + 1) * ch, ch)],
                    bufs[1 - b], rsem.at[1 - b])
        w[nch - 1].wait()

    return pl.kernel(
        body,
        out_type=jax.ShapeDtypeStruct((V, D), rows.dtype),
        mesh=_sc_mesh(),
        scratch_types=[
            pltpu.VMEM((nch, ch), jnp.int32),
            pltpu.VMEM((ch, D), rows.dtype),
            pltpu.VMEM((ch, D), rows.dtype),
            pltpu.SemaphoreType.DMA((2,)),
            pltpu.SemaphoreType.DMA((2,)),
        ],
    )(rows, idx2)


def kernel(x, Wg, bg, W1, b1, W2, b2):
    B, S, D = x.shape
    E = Wg.shape[1]
    N = B * S
    NT = N // _T + E
    NPAD = NT * _T

    xf = x.reshape(N, D)
    scores = _gate(xf, Wg, bg)
    s8 = scores[:, :E]
    eid = jnp.argmax(s8, axis=1).astype(jnp.int32)
    oh = (eid[:, None] == jnp.arange(E, dtype=jnp.int32)[None, :]).astype(
        jnp.int32)
    cc = jnp.cumsum(oh, axis=0)
    counts = cc[-1]
    grank = jnp.take_along_axis(cc, eid[:, None], axis=1)[:, 0] - 1

    tiles_per_e = (counts + _T - 1) // _T
    tile_bound = jnp.cumsum(tiles_per_e).astype(jnp.int32)
    pad_off = jnp.concatenate([jnp.zeros((1,), jnp.int32),
                               tile_bound[:-1]]) * _T
    pos = pad_off[eid] + grank
    t_idx = jnp.arange(NT, dtype=jnp.int32)
    tile_eid = jnp.sum((t_idx[:, None] >= tile_bound[None, :]).astype(
        jnp.int32), axis=1)
    valid = (tile_eid < E).astype(jnp.int32)
    pref = jnp.stack([jnp.minimum(tile_eid, E - 1), valid])

    xs = _sc_scatter(xf, pos, NPAD)
    ys = _ffn(xs, pref, W1, b1, W2, b2)
    out = _sc_gather(ys, pos)
    return (out.reshape(B, S, D), s8.reshape(B, S, E))


# speedup vs baseline: 1.0916x; 1.0461x over previous
import jax
import jax.numpy as jnp
from jax import lax
from jax.experimental import pallas as pl
from jax.experimental.pallas import tpu as pltpu
from jax.experimental.pallas import tpu_sc as plsc

_T = 512
_TG = 512
_NW = 32


def _gate_body(x_ref, wg_ref, bg_ref, s_ref):
    s_ref[...] = jnp.dot(x_ref[...], wg_ref[...],
                         preferred_element_type=jnp.float32) + bg_ref[...]


def _gate(xf, Wg, bg):
    N, D = xf.shape
    E = Wg.shape[1]
    NB = N // _TG
    wg_pad = jnp.zeros((D, 128), Wg.dtype).at[:, :E].set(Wg)
    bg_pad = jnp.zeros((1, 128), jnp.float32).at[0, :E].set(bg)
    return pl.pallas_call(
        _gate_body,
        grid=(NB,),
        in_specs=[
            pl.BlockSpec((_TG, D), lambda i: (i, 0)),
            pl.BlockSpec((D, 128), lambda i: (0, 0)),
            pl.BlockSpec((1, 128), lambda i: (0, 0)),
        ],
        out_specs=pl.BlockSpec((_TG, 128), lambda i: (i, 0)),
        out_shape=jax.ShapeDtypeStruct((N, 128), jnp.float32),
    )(xf, wg_pad, bg_pad)


def _ffn_body(pref_ref, xs_ref, w1_ref, b1_ref, w2_ref, b2_ref, y_ref):
    @pl.when(pref_ref[1, pl.program_id(0)] == 1)
    def _():
        xv = xs_ref[...].astype(jnp.bfloat16)
        h = jnp.dot(xv, w1_ref[0].astype(jnp.bfloat16),
                    preferred_element_type=jnp.float32)
        h = jnp.maximum(h + b1_ref[0, 0], 0.0)
        y = jnp.dot(h.astype(jnp.bfloat16), w2_ref[0].astype(jnp.bfloat16),
                    preferred_element_type=jnp.float32)
        y_ref[...] = y + b2_ref[0, 0]


def _ffn(xs, pref, W1, b1, W2, b2):
    NPAD, D = xs.shape
    E, _, H = W1.shape
    NT = NPAD // _T
    grid_spec = pltpu.PrefetchScalarGridSpec(
        num_scalar_prefetch=1,
        grid=(NT,),
        in_specs=[
            pl.BlockSpec((_T, D), lambda i, p: (i, 0)),
            pl.BlockSpec((1, D, H), lambda i, p: (p[0, i], 0, 0)),
            pl.BlockSpec((1, 1, H), lambda i, p: (p[0, i], 0, 0)),
            pl.BlockSpec((1, H, D), lambda i, p: (p[0, i], 0, 0)),
            pl.BlockSpec((1, 1, D), lambda i, p: (p[0, i], 0, 0)),
        ],
        out_specs=pl.BlockSpec((_T, D), lambda i, p: (i, 0)),
    )
    return pl.pallas_call(
        _ffn_body,
        grid_spec=grid_spec,
        out_shape=jax.ShapeDtypeStruct((NPAD, D), jnp.float32),
    )(pref, xs, W1, b1.reshape(E, 1, H), W2, b2.reshape(E, 1, D))


def _sc_mesh():
    return plsc.VectorSubcoreMesh(core_axis_name="c", subcore_axis_name="s")


def _sc_gather(table, idx):
    V, D = table.shape
    B = idx.shape[0]
    bpw = B // _NW
    nch = 4
    ch = bpw // nch

    def body(tab_hbm, idx_hbm, out_hbm, idx_v, b0, b1, gsem, osem):
        wid = lax.axis_index("s") * 2 + lax.axis_index("c")
        base = wid * bpw
        pltpu.sync_copy(idx_hbm.at[pl.ds(base, bpw)], idx_v)
        bufs = (b0, b1)
        g = [None] * nch
        o = [None] * nch
        g[0] = pltpu.async_copy(tab_hbm.at[idx_v.at[pl.ds(0, ch)]], bufs[0],
                                gsem.at[0])
        for c in range(nch):
            b = c % 2
            g[c].wait()
            if c >= 1:
                o[c - 1].wait()
            o[c] = pltpu.async_copy(bufs[b], out_hbm.at[pl.ds(base + c * ch, ch)],
                                    osem.at[b])
            if c + 1 < nch:
                g[c + 1] = pltpu.async_copy(
                    tab_hbm.at[idx_v.at[pl.ds((c + 1) * ch, ch)]],
                    bufs[1 - b], gsem.at[1 - b])
        o[nch - 1].wait()

    return pl.kernel(
        body,
        out_type=jax.ShapeDtypeStruct((B, D), table.dtype),
        mesh=_sc_mesh(),
        scratch_types=[
            pltpu.VMEM((bpw,), jnp.int32),
            pltpu.VMEM((ch, D), table.dtype),
            pltpu.VMEM((ch, D), table.dtype),
            pltpu.SemaphoreType.DMA((2,)),
            pltpu.SemaphoreType.DMA((2,)),
        ],
    )(table, idx)


def _sc_scatter(rows, idx, V):
    B, D = rows.shape
    bpw = B // _NW
    nch = 4
    ch = bpw // nch
    idx2 = idx.reshape(_NW * nch, ch)

    def body(rows_hbm, idx_hbm, out_hbm, idx_v, b0, b1, rsem, wsem):
        wid = lax.axis_index("s") * 2 + lax.axis_index("c")
        base = wid * bpw
        pltpu.sync_copy(idx_hbm.at[pl.ds(wid * nch, nch)], idx_v)
        bufs = (b0, b1)
        r = [None] * nch
        w = [None] * nch
        r[0] = pltpu.async_copy(rows_hbm.at[pl.ds(base, ch)], bufs[0],
                                rsem.at[0])
        for c in range(nch):
            b = c % 2
            r[c].wait()
            if c >= 1:
                w[c - 1].wait()
            w[c] = pltpu.async_copy(bufs[b], out_hbm.at[idx_v.at[c]],
                                    wsem.at[b])
            if c + 1 < nch:
                r[c + 1] = pltpu.async_copy(
                    rows_hbm.at[pl.ds(base + (c + 1) * ch, ch)],
                    bufs[1 - b], rsem.at[1 - b])
        w[nch - 1].wait()

    return pl.kernel(
        body,
        out_type=jax.ShapeDtypeStruct((V, D), rows.dtype),
        mesh=_sc_mesh(),
        scratch_types=[
            pltpu.VMEM((nch, ch), jnp.int32),
            pltpu.VMEM((ch, D), rows.dtype),
            pltpu.VMEM((ch, D), rows.dtype),
            pltpu.SemaphoreType.DMA((2,)),
            pltpu.SemaphoreType.DMA((2,)),
        ],
    )(rows, idx2)


def kernel(x, Wg, bg, W1, b1, W2, b2):
    B, S, D = x.shape
    E = Wg.shape[1]
    N = B * S
    NT = N // _T + E
    NPAD = NT * _T

    xf = x.reshape(N, D)
    scores = _gate(xf, Wg, bg)
    s8 = scores[:, :E]
    eid = jnp.argmax(s8, axis=1).astype(jnp.int32)
    oh = (eid[:, None] == jnp.arange(E, dtype=jnp.int32)[None, :]).astype(
        jnp.int32)
    cc = jnp.cumsum(oh, axis=0)
    counts = cc[-1]
    grank = jnp.take_along_axis(cc, eid[:, None], axis=1)[:, 0] - 1

    tiles_per_e = (counts + _T - 1) // _T
    tile_bound = jnp.cumsum(tiles_per_e).astype(jnp.int32)
    pad_off = jnp.concatenate([jnp.zeros((1,), jnp.int32),
                               tile_bound[:-1]]) * _T
    pos = pad_off[eid] + grank
    t_idx = jnp.arange(NT, dtype=jnp.int32)
    tile_eid = jnp.sum((t_idx[:, None] >= tile_bound[None, :]).astype(
        jnp.int32), axis=1)
    valid = (tile_eid < E).astype(jnp.int32)
    pref = jnp.stack([jnp.minimum(tile_eid, E - 1), valid])

    xs = _sc_scatter(xf, pos, NPAD)
    ys = _ffn(xs, pref, W1, b1, W2, b2)
    out = _sc_gather(ys, pos)
    return (out.reshape(B, S, D), s8.reshape(B, S, E))
